# baseline (speedup 1.0000x reference)
import functools

import jax
import jax.numpy as jnp
from jax import lax
from jax.experimental import pallas as pl
from jax.experimental.pallas import tpu as pltpu
from jax.experimental.pallas import tpu_sc as plsc

N1 = 32768
N2 = 8192
B = 4
DF = 64
DC = 64
C = N2 // B
R = 512
BLOCKS_PER_CLOUD = (N1 // B) // R


def _nn_body(xyz1t_ref, xyz2_ref, i0, i1, i2, w0, w1, w2):
    xft = xyz1t_ref[...]
    xc = xyz2_ref[...]
    sqf = jnp.sum(xft * xft, axis=0, keepdims=True)
    sqc = jnp.sum(xc * xc, axis=1, keepdims=True)
    prod = jnp.dot(xc, xft, preferred_element_type=jnp.float32)
    d2 = (sqf + sqc) - 2.0 * prod
    iota = lax.broadcasted_iota(jnp.int32, d2.shape, 0)
    ms, isels = [], []
    for k in range(3):
        m = jnp.min(d2, axis=0, keepdims=True)
        isel = jnp.min(jnp.where(d2 == m, iota, C), axis=0, keepdims=True)
        ms.append(m)
        isels.append(isel)
        if k < 2:
            d2 = jnp.where(iota == isel, jnp.float32(jnp.inf), d2)
    dist = jnp.maximum(jnp.concatenate(ms, axis=0), 0.0)
    recip = 1.0 / (dist + 1e-8)
    w = recip / jnp.sum(recip, axis=0, keepdims=True)
    cloud = pl.program_id(0) // BLOCKS_PER_CLOUD
    for k, ref in enumerate((i0, i1, i2)):
        ref[...] = (isels[k] + cloud * C).reshape(R)
    for k, ref in enumerate((w0, w1, w2)):
        ref[...] = w[k, :].reshape(R)


def _phase1(xyz1, xyz2, interpret=False):
    return pl.pallas_call(
        _nn_body,
        grid=(N1 // R,),
        in_specs=[
            pl.BlockSpec((3, R), lambda i: (0, i)),
            pl.BlockSpec((C, 3), lambda i: (i // BLOCKS_PER_CLOUD, 0)),
        ],
        out_specs=[pl.BlockSpec((R,), lambda i: (i,))] * 6,
        out_shape=[jax.ShapeDtypeStruct((N1,), jnp.int32)] * 3
        + [jax.ShapeDtypeStruct((N1,), jnp.float32)] * 3,
        interpret=interpret,
    )(xyz1, xyz2)


_SC_NUM_CORES = 2
_SC_NUM_SUBCORES = 16
NW = _SC_NUM_CORES * _SC_NUM_SUBCORES
PW = N1 // NW
CH = 64
NCH = PW // CH


def _interp_sc_body(ih0, ih1, ih2, wh0, wh1, wh2, feat2_hbm, out_hbm,
                    ia0, ia1, ia2, ib0, ib1, ib2,
                    wa0, wa1, wa2, wb0, wb1, wb2,
                    ra0, ra1, ra2, rb0, rb1, rb2,
                    aa, ab, gs0, gs1, os0, os1):
    wid = lax.axis_index("s") * _SC_NUM_CORES + lax.axis_index("c")
    base = wid * PW
    ihs = (ih0, ih1, ih2)
    whs = (wh0, wh1, wh2)
    idx_v = ((ia0, ia1, ia2), (ib0, ib1, ib2))
    w_v = ((wa0, wa1, wa2), (wb0, wb1, wb2))
    rows_v = ((ra0, ra1, ra2), (rb0, rb1, rb2))
    acc_v = (aa, ab)
    gsems = (gs0, gs1)
    osems = (os0, os1)

    def issue(ci):
        b = ci % 2
        cps = []
        for k in range(3):
            pltpu.sync_copy(ihs[k].at[pl.ds(base + ci * CH, CH)],
                            idx_v[b][k])
            pltpu.sync_copy(whs[k].at[pl.ds(base + ci * CH, CH)],
                            w_v[b][k].at[pl.ds(0, CH)])
            cps.append(pltpu.async_copy(
                feat2_hbm.at[idx_v[b][k]], rows_v[b][k], gsems[b]))
        return cps

    pending = {0: issue(0)}
    out_cps = {}
    for ci in range(NCH):
        b = ci % 2
        if ci + 1 < NCH:
            pending[ci + 1] = issue(ci + 1)
        for cp in pending.pop(ci):
            cp.wait()
        if ci >= 2:
            out_cps.pop(ci - 2).wait()

        def point_body(p, _, b=b):
            wk = [jnp.full((16,), w_v[b][k][pl.ds(p, 16)][0], jnp.float32)
                  for k in range(3)]
            for cc in range(DC // 16):
                sl = pl.ds(cc * 16, 16)
                acc_v[b][p, sl] = (wk[0] * rows_v[b][0][p, sl]
                                   + wk[1] * rows_v[b][1][p, sl]
                                   + wk[2] * rows_v[b][2][p, sl])
            return 0

        lax.fori_loop(0, CH, point_body, 0, unroll=2)
        out_cps[ci] = pltpu.async_copy(
            acc_v[b], out_hbm.at[pl.ds(base + ci * CH, CH), :], osems[b])
    for cp in out_cps.values():
        cp.wait()


def _phase2_sc(i0, i1, i2, w0, w1, w2, feature2):
    mesh = plsc.VectorSubcoreMesh(core_axis_name="c", subcore_axis_name="s")
    fn = functools.partial(
        pl.kernel,
        mesh=mesh,
        out_type=jax.ShapeDtypeStruct((N1, DC), jnp.float32),
        scratch_types=(
            [pltpu.VMEM((CH,), jnp.int32)] * 6
            + [pltpu.VMEM((CH + 16,), jnp.float32)] * 6
            + [pltpu.VMEM((CH, 128), jnp.float32)] * 6
            + [pltpu.VMEM((CH, DC), jnp.float32)] * 2
            + [pltpu.SemaphoreType.DMA] * 4
        ),
    )(_interp_sc_body)
    return fn(i0, i1, i2, w0, w1, w2, feature2)


def _ln_relu(x, g, b):
    mu = jnp.mean(x, axis=1, keepdims=True)
    var = jnp.mean((x - mu) ** 2, axis=1, keepdims=True)
    return jnp.maximum((x - mu) / jnp.sqrt(var + 1e-5) * g + b, 0.0)


def _mlp_body(f1_ref, it_ref, w1_ref, b1_ref, g1_ref, be1_ref,
              w2_ref, b2_ref, g2_ref, be2_ref, o_ref):
    f1 = f1_ref[...]
    it = it_ref[...]
    x = (jnp.dot(f1, w1_ref[0:DF, :], preferred_element_type=jnp.float32)
         + jnp.dot(it, w1_ref[DF:DF + DC, :], preferred_element_type=jnp.float32)
         + b1_ref[...])
    x = _ln_relu(x, g1_ref[...], be1_ref[...])
    x = jnp.dot(x, w2_ref[...], preferred_element_type=jnp.float32) + b2_ref[...]
    o_ref[...] = _ln_relu(x, g2_ref[...], be2_ref[...])


def _phase3(feature1, interp, W1, b1, g1, be1, W2, b2, g2, be2,
            interpret=False):
    full = lambda a, b: pl.BlockSpec((a, b), lambda i: (0, 0))
    return pl.pallas_call(
        _mlp_body,
        grid=(N1 // R,),
        in_specs=[
            pl.BlockSpec((R, DF), lambda i: (i, 0)),
            pl.BlockSpec((R, DC), lambda i: (i, 0)),
            full(DF + DC, 64), full(1, 64), full(1, 64), full(1, 64),
            full(64, 64), full(1, 64), full(1, 64), full(1, 64),
        ],
        out_specs=pl.BlockSpec((R, 64), lambda i: (i, 0)),
        out_shape=jax.ShapeDtypeStruct((N1, 64), jnp.float32),
        interpret=interpret,
    )(feature1, interp, W1, b1.reshape(1, -1), g1.reshape(1, -1),
      be1.reshape(1, -1), W2, b2.reshape(1, -1), g2.reshape(1, -1),
      be2.reshape(1, -1))


def kernel(xyz1, xyz2, feature1, feature2, offset1, offset2,
           W1, b1, g1, be1, W2, b2, g2, be2):
    i0, i1, i2, w0, w1, w2 = _phase1(xyz1.T, xyz2)
    feature2p = jnp.pad(feature2, ((0, 0), (0, 128 - DC)))
    interp = _phase2_sc(i0, i1, i2, w0, w1, w2, feature2p)
    return _phase3(feature1, interp, W1, b1, g1, be1, W2, b2, g2, be2)


# speedup vs baseline: 1.2026x; 1.2026x over previous
import functools

import jax
import jax.numpy as jnp
from jax import lax
from jax.experimental import pallas as pl
from jax.experimental.pallas import tpu as pltpu
from jax.experimental.pallas import tpu_sc as plsc

N1 = 32768
N2 = 8192
B = 4
DF = 64
DC = 64
C = N2 // B
R = 1024
BLOCKS_PER_CLOUD = (N1 // B) // R


def _nn_body(xyz1t_ref, xyz2_ref, i0, i1, i2, w0, w1, w2):
    xft = xyz1t_ref[...]
    xc = xyz2_ref[...]
    sqf = jnp.sum(xft * xft, axis=0, keepdims=True)
    sqc = jnp.sum(xc * xc, axis=1, keepdims=True)
    prod = jnp.dot(xc, xft, preferred_element_type=jnp.float32)
    d2 = (sqf + sqc) - 2.0 * prod
    iota = lax.broadcasted_iota(jnp.int32, d2.shape, 0)
    ms, isels = [], []
    for k in range(3):
        m = jnp.min(d2, axis=0, keepdims=True)
        isel = jnp.min(jnp.where(d2 == m, iota, C), axis=0, keepdims=True)
        ms.append(m)
        isels.append(isel)
        if k < 2:
            d2 = jnp.where(iota == isel, jnp.float32(jnp.inf), d2)
    dist = jnp.maximum(jnp.concatenate(ms, axis=0), 0.0)
    recip = 1.0 / (dist + 1e-8)
    w = recip / jnp.sum(recip, axis=0, keepdims=True)
    cloud = pl.program_id(0) // BLOCKS_PER_CLOUD
    for k, ref in enumerate((i0, i1, i2)):
        ref[...] = (isels[k] + cloud * C).reshape(R)
    for k, ref in enumerate((w0, w1, w2)):
        ref[...] = w[k, :].reshape(R)


def _phase1(xyz1, xyz2, interpret=False):
    return pl.pallas_call(
        _nn_body,
        grid=(N1 // R,),
        in_specs=[
            pl.BlockSpec((3, R), lambda i: (0, i)),
            pl.BlockSpec((C, 3), lambda i: (i // BLOCKS_PER_CLOUD, 0)),
        ],
        out_specs=[pl.BlockSpec((R,), lambda i: (i,))] * 6,
        out_shape=[jax.ShapeDtypeStruct((N1,), jnp.int32)] * 3
        + [jax.ShapeDtypeStruct((N1,), jnp.float32)] * 3,
        interpret=interpret,
    )(xyz1, xyz2)


_SC_NUM_CORES = 2
_SC_NUM_SUBCORES = 16
NW = _SC_NUM_CORES * _SC_NUM_SUBCORES
PW = N1 // NW
CH = 128
NCH = PW // CH


def _interp_sc_body(ih0, ih1, ih2, wh0, wh1, wh2, feat2_hbm, out_hbm,
                    i0, i1, i2, w0, w1, w2, r0, r1, r2, acc_v,
                    s0, s1, s2, osem):
    wid = lax.axis_index("s") * _SC_NUM_CORES + lax.axis_index("c")
    base = wid * PW
    ihs = (ih0, ih1, ih2)
    whs = (wh0, wh1, wh2)
    idx_vs = (i0, i1, i2)
    w_vs = (w0, w1, w2)
    row_vs = (r0, r1, r2)
    sems = (s0, s1, s2)

    def chunk_body(ci, _):
        pbase = base + ci * CH
        copies = []
        for k in range(3):
            pltpu.sync_copy(ihs[k].at[pl.ds(pbase, CH)], idx_vs[k])
            pltpu.sync_copy(whs[k].at[pl.ds(pbase, CH)],
                            w_vs[k].at[pl.ds(0, CH)])
            copies.append(pltpu.async_copy(
                feat2_hbm.at[idx_vs[k]], row_vs[k], sems[k]))
        for cp in copies:
            cp.wait()

        def point_body(p, _):
            wk = [jnp.full((16,), w_vs[k][pl.ds(p, 16)][0], jnp.float32)
                  for k in range(3)]
            for cc in range(DC // 16):
                sl = pl.ds(cc * 16, 16)
                acc = wk[0] * r0[p, sl] + wk[1] * r1[p, sl] + wk[2] * r2[p, sl]
                acc_v[p, sl] = acc
            return 0

        lax.fori_loop(0, CH, point_body, 0)
        pltpu.async_copy(acc_v, out_hbm.at[pl.ds(pbase, CH), :], osem).wait()
        return 0

    lax.fori_loop(0, NCH, chunk_body, 0)


def _phase2_sc(i0, i1, i2, w0, w1, w2, feature2):
    mesh = plsc.VectorSubcoreMesh(core_axis_name="c", subcore_axis_name="s")
    fn = functools.partial(
        pl.kernel,
        mesh=mesh,
        out_type=jax.ShapeDtypeStruct((N1, DC), jnp.float32),
        scratch_types=(
            [pltpu.VMEM((CH,), jnp.int32)] * 3
            + [pltpu.VMEM((CH + 16,), jnp.float32)] * 3
            + [pltpu.VMEM((CH, 128), jnp.float32)] * 3
            + [pltpu.VMEM((CH, DC), jnp.float32)]
            + [pltpu.SemaphoreType.DMA] * 4
        ),
    )(_interp_sc_body)
    return fn(i0, i1, i2, w0, w1, w2, feature2)


def _ln_relu(x, g, b):
    mu = jnp.mean(x, axis=1, keepdims=True)
    var = jnp.mean((x - mu) ** 2, axis=1, keepdims=True)
    return jnp.maximum((x - mu) / jnp.sqrt(var + 1e-5) * g + b, 0.0)


def _mlp_body(f1_ref, it_ref, w1_ref, b1_ref, g1_ref, be1_ref,
              w2_ref, b2_ref, g2_ref, be2_ref, o_ref):
    f1 = f1_ref[...]
    it = it_ref[...]
    x = (jnp.dot(f1, w1_ref[0:DF, :], preferred_element_type=jnp.float32)
         + jnp.dot(it, w1_ref[DF:DF + DC, :], preferred_element_type=jnp.float32)
         + b1_ref[...])
    x = _ln_relu(x, g1_ref[...], be1_ref[...])
    x = jnp.dot(x, w2_ref[...], preferred_element_type=jnp.float32) + b2_ref[...]
    o_ref[...] = _ln_relu(x, g2_ref[...], be2_ref[...])


def _phase3(feature1, interp, W1, b1, g1, be1, W2, b2, g2, be2,
            interpret=False):
    full = lambda a, b: pl.BlockSpec((a, b), lambda i: (0, 0))
    return pl.pallas_call(
        _mlp_body,
        grid=(N1 // R,),
        in_specs=[
            pl.BlockSpec((R, DF), lambda i: (i, 0)),
            pl.BlockSpec((R, DC), lambda i: (i, 0)),
            full(DF + DC, 64), full(1, 64), full(1, 64), full(1, 64),
            full(64, 64), full(1, 64), full(1, 64), full(1, 64),
        ],
        out_specs=pl.BlockSpec((R, 64), lambda i: (i, 0)),
        out_shape=jax.ShapeDtypeStruct((N1, 64), jnp.float32),
        interpret=interpret,
    )(feature1, interp, W1, b1.reshape(1, -1), g1.reshape(1, -1),
      be1.reshape(1, -1), W2, b2.reshape(1, -1), g2.reshape(1, -1),
      be2.reshape(1, -1))


def kernel(xyz1, xyz2, feature1, feature2, offset1, offset2,
           W1, b1, g1, be1, W2, b2, g2, be2):
    i0, i1, i2, w0, w1, w2 = _phase1(xyz1.T, xyz2)
    feature2p = jnp.pad(feature2, ((0, 0), (0, 128 - DC)))
    interp = _phase2_sc(i0, i1, i2, w0, w1, w2, feature2p)
    return _phase3(feature1, interp, W1, b1, g1, be1, W2, b2, g2, be2)


# speedup vs baseline: 1.2574x; 1.0455x over previous
import functools

import jax
import jax.numpy as jnp
from jax import lax
from jax.experimental import pallas as pl
from jax.experimental.pallas import tpu as pltpu
from jax.experimental.pallas import tpu_sc as plsc

N1 = 32768
N2 = 8192
B = 4
DF = 64
DC = 64
C = N2 // B
R = 2048
BLOCKS_PER_CLOUD = (N1 // B) // R


def _nn_body(xyz1t_ref, xyz2_ref, i0, i1, i2, w0, w1, w2):
    xft = xyz1t_ref[...]
    xc = xyz2_ref[...]
    sqf = jnp.sum(xft * xft, axis=0, keepdims=True)
    sqc = jnp.sum(xc * xc, axis=1, keepdims=True)
    prod = jnp.dot(xc, xft, preferred_element_type=jnp.float32)
    d2 = (sqf + sqc) - 2.0 * prod
    iota = lax.broadcasted_iota(jnp.int32, d2.shape, 0)
    ms, isels = [], []
    for k in range(3):
        m = jnp.min(d2, axis=0, keepdims=True)
        isel = jnp.min(jnp.where(d2 == m, iota, C), axis=0, keepdims=True)
        ms.append(m)
        isels.append(isel)
        if k < 2:
            d2 = jnp.where(iota == isel, jnp.float32(jnp.inf), d2)
    dist = jnp.maximum(jnp.concatenate(ms, axis=0), 0.0)
    recip = 1.0 / (dist + 1e-8)
    w = recip / jnp.sum(recip, axis=0, keepdims=True)
    cloud = pl.program_id(0) // BLOCKS_PER_CLOUD
    for k, ref in enumerate((i0, i1, i2)):
        ref[...] = (isels[k] + cloud * C).reshape(R)
    for k, ref in enumerate((w0, w1, w2)):
        ref[...] = w[k, :].reshape(R)


def _phase1(xyz1, xyz2, interpret=False):
    return pl.pallas_call(
        _nn_body,
        grid=(N1 // R,),
        in_specs=[
            pl.BlockSpec((3, R), lambda i: (0, i)),
            pl.BlockSpec((C, 3), lambda i: (i // BLOCKS_PER_CLOUD, 0)),
        ],
        out_specs=[pl.BlockSpec((R,), lambda i: (i,))] * 6,
        out_shape=[jax.ShapeDtypeStruct((N1,), jnp.int32)] * 3
        + [jax.ShapeDtypeStruct((N1,), jnp.float32)] * 3,
        interpret=interpret,
    )(xyz1, xyz2)


_SC_NUM_CORES = 2
_SC_NUM_SUBCORES = 16
NW = _SC_NUM_CORES * _SC_NUM_SUBCORES
PW = N1 // NW
CH = 128
NCH = PW // CH


def _interp_sc_body(ih0, ih1, ih2, wh0, wh1, wh2, feat2_hbm, out_hbm,
                    i0, i1, i2, w0, w1, w2, r0, r1, r2, acc_v,
                    s0, s1, s2, osem):
    wid = lax.axis_index("s") * _SC_NUM_CORES + lax.axis_index("c")
    base = wid * PW
    ihs = (ih0, ih1, ih2)
    whs = (wh0, wh1, wh2)
    idx_vs = (i0, i1, i2)
    w_vs = (w0, w1, w2)
    row_vs = (r0, r1, r2)
    sems = (s0, s1, s2)

    def chunk_body(ci, _):
        pbase = base + ci * CH
        copies = []
        for k in range(3):
            pltpu.sync_copy(ihs[k].at[pl.ds(pbase, CH)], idx_vs[k])
            pltpu.sync_copy(whs[k].at[pl.ds(pbase, CH)],
                            w_vs[k].at[pl.ds(0, CH)])
            copies.append(pltpu.async_copy(
                feat2_hbm.at[idx_vs[k]], row_vs[k], sems[k]))
        for cp in copies:
            cp.wait()

        def point_body(p, _):
            wk = [jnp.full((16,), w_vs[k][pl.ds(p, 16)][0], jnp.float32)
                  for k in range(3)]
            for cc in range(DC // 16):
                sl = pl.ds(cc * 16, 16)
                acc = wk[0] * r0[p, sl] + wk[1] * r1[p, sl] + wk[2] * r2[p, sl]
                acc_v[p, sl] = acc
            return 0

        lax.fori_loop(0, CH, point_body, 0)
        pltpu.async_copy(acc_v, out_hbm.at[pl.ds(pbase, CH), :], osem).wait()
        return 0

    lax.fori_loop(0, NCH, chunk_body, 0)


def _phase2_sc(i0, i1, i2, w0, w1, w2, feature2):
    mesh = plsc.VectorSubcoreMesh(core_axis_name="c", subcore_axis_name="s")
    fn = functools.partial(
        pl.kernel,
        mesh=mesh,
        out_type=jax.ShapeDtypeStruct((N1, DC), jnp.float32),
        scratch_types=(
            [pltpu.VMEM((CH,), jnp.int32)] * 3
            + [pltpu.VMEM((CH + 16,), jnp.float32)] * 3
            + [pltpu.VMEM((CH, 128), jnp.float32)] * 3
            + [pltpu.VMEM((CH, DC), jnp.float32)]
            + [pltpu.SemaphoreType.DMA] * 4
        ),
    )(_interp_sc_body)
    return fn(i0, i1, i2, w0, w1, w2, feature2)


def _ln_relu(x, g, b):
    mu = jnp.mean(x, axis=1, keepdims=True)
    var = jnp.mean((x - mu) ** 2, axis=1, keepdims=True)
    return jnp.maximum((x - mu) / jnp.sqrt(var + 1e-5) * g + b, 0.0)


def _mlp_body(f1_ref, it_ref, w1_ref, b1_ref, g1_ref, be1_ref,
              w2_ref, b2_ref, g2_ref, be2_ref, o_ref):
    f1 = f1_ref[...]
    it = it_ref[...]
    x = (jnp.dot(f1, w1_ref[0:DF, :], preferred_element_type=jnp.float32)
         + jnp.dot(it, w1_ref[DF:DF + DC, :], preferred_element_type=jnp.float32)
         + b1_ref[...])
    x = _ln_relu(x, g1_ref[...], be1_ref[...])
    x = jnp.dot(x, w2_ref[...], preferred_element_type=jnp.float32) + b2_ref[...]
    o_ref[...] = _ln_relu(x, g2_ref[...], be2_ref[...])


def _phase3(feature1, interp, W1, b1, g1, be1, W2, b2, g2, be2,
            interpret=False):
    full = lambda a, b: pl.BlockSpec((a, b), lambda i: (0, 0))
    return pl.pallas_call(
        _mlp_body,
        grid=(N1 // R,),
        in_specs=[
            pl.BlockSpec((R, DF), lambda i: (i, 0)),
            pl.BlockSpec((R, DC), lambda i: (i, 0)),
            full(DF + DC, 64), full(1, 64), full(1, 64), full(1, 64),
            full(64, 64), full(1, 64), full(1, 64), full(1, 64),
        ],
        out_specs=pl.BlockSpec((R, 64), lambda i: (i, 0)),
        out_shape=jax.ShapeDtypeStruct((N1, 64), jnp.float32),
        interpret=interpret,
    )(feature1, interp, W1, b1.reshape(1, -1), g1.reshape(1, -1),
      be1.reshape(1, -1), W2, b2.reshape(1, -1), g2.reshape(1, -1),
      be2.reshape(1, -1))


def kernel(xyz1, xyz2, feature1, feature2, offset1, offset2,
           W1, b1, g1, be1, W2, b2, g2, be2):
    i0, i1, i2, w0, w1, w2 = _phase1(xyz1.T, xyz2)
    feature2p = jnp.pad(feature2, ((0, 0), (0, 128 - DC)))
    interp = _phase2_sc(i0, i1, i2, w0, w1, w2, feature2p)
    return _phase3(feature1, interp, W1, b1, g1, be1, W2, b2, g2, be2)


# speedup vs baseline: 1.2822x; 1.0197x over previous
import functools

import jax
import jax.numpy as jnp
from jax import lax
from jax.experimental import pallas as pl
from jax.experimental.pallas import tpu as pltpu
from jax.experimental.pallas import tpu_sc as plsc

N1 = 32768
N2 = 8192
B = 4
DF = 64
DC = 64
C = N2 // B
R = 4096
BLOCKS_PER_CLOUD = (N1 // B) // R


def _nn_body(xyz1t_ref, xyz2_ref, i0, i1, i2, w0, w1, w2):
    xft = xyz1t_ref[...]
    xc = xyz2_ref[...]
    sqf = jnp.sum(xft * xft, axis=0, keepdims=True)
    sqc = jnp.sum(xc * xc, axis=1, keepdims=True)
    prod = jnp.dot(xc, xft, preferred_element_type=jnp.float32)
    d2 = (sqf + sqc) - 2.0 * prod
    iota = lax.broadcasted_iota(jnp.int32, d2.shape, 0)
    ms, isels = [], []
    for k in range(3):
        m = jnp.min(d2, axis=0, keepdims=True)
        isel = jnp.min(jnp.where(d2 == m, iota, C), axis=0, keepdims=True)
        ms.append(m)
        isels.append(isel)
        if k < 2:
            d2 = jnp.where(iota == isel, jnp.float32(jnp.inf), d2)
    dist = jnp.maximum(jnp.concatenate(ms, axis=0), 0.0)
    recip = 1.0 / (dist + 1e-8)
    w = recip / jnp.sum(recip, axis=0, keepdims=True)
    cloud = pl.program_id(0) // BLOCKS_PER_CLOUD
    for k, ref in enumerate((i0, i1, i2)):
        ref[...] = (isels[k] + cloud * C).reshape(R)
    for k, ref in enumerate((w0, w1, w2)):
        ref[...] = w[k, :].reshape(R)


def _phase1(xyz1, xyz2, interpret=False):
    return pl.pallas_call(
        _nn_body,
        grid=(N1 // R,),
        in_specs=[
            pl.BlockSpec((3, R), lambda i: (0, i)),
            pl.BlockSpec((C, 3), lambda i: (i // BLOCKS_PER_CLOUD, 0)),
        ],
        out_specs=[pl.BlockSpec((R,), lambda i: (i,))] * 6,
        out_shape=[jax.ShapeDtypeStruct((N1,), jnp.int32)] * 3
        + [jax.ShapeDtypeStruct((N1,), jnp.float32)] * 3,
        interpret=interpret,
    )(xyz1, xyz2)


_SC_NUM_CORES = 2
_SC_NUM_SUBCORES = 16
NW = _SC_NUM_CORES * _SC_NUM_SUBCORES
PW = N1 // NW
CH = 128
NCH = PW // CH


def _interp_sc_body(ih0, ih1, ih2, wh0, wh1, wh2, feat2_hbm, out_hbm,
                    i0, i1, i2, w0, w1, w2, r0, r1, r2, acc_v,
                    s0, s1, s2, osem):
    wid = lax.axis_index("s") * _SC_NUM_CORES + lax.axis_index("c")
    base = wid * PW
    ihs = (ih0, ih1, ih2)
    whs = (wh0, wh1, wh2)
    idx_vs = (i0, i1, i2)
    w_vs = (w0, w1, w2)
    row_vs = (r0, r1, r2)
    sems = (s0, s1, s2)

    def chunk_body(ci, _):
        pbase = base + ci * CH
        copies = []
        for k in range(3):
            pltpu.sync_copy(ihs[k].at[pl.ds(pbase, CH)], idx_vs[k])
            pltpu.sync_copy(whs[k].at[pl.ds(pbase, CH)],
                            w_vs[k].at[pl.ds(0, CH)])
            copies.append(pltpu.async_copy(
                feat2_hbm.at[idx_vs[k]], row_vs[k], sems[k]))
        for cp in copies:
            cp.wait()

        def point_body(p, _):
            wk = [jnp.full((16,), w_vs[k][pl.ds(p, 16)][0], jnp.float32)
                  for k in range(3)]
            for cc in range(DC // 16):
                sl = pl.ds(cc * 16, 16)
                acc = wk[0] * r0[p, sl] + wk[1] * r1[p, sl] + wk[2] * r2[p, sl]
                acc_v[p, sl] = acc
            return 0

        lax.fori_loop(0, CH, point_body, 0)
        pltpu.async_copy(acc_v, out_hbm.at[pl.ds(pbase, CH), :], osem).wait()
        return 0

    lax.fori_loop(0, NCH, chunk_body, 0)


def _phase2_sc(i0, i1, i2, w0, w1, w2, feature2):
    mesh = plsc.VectorSubcoreMesh(core_axis_name="c", subcore_axis_name="s")
    fn = functools.partial(
        pl.kernel,
        mesh=mesh,
        out_type=jax.ShapeDtypeStruct((N1, DC), jnp.float32),
        scratch_types=(
            [pltpu.VMEM((CH,), jnp.int32)] * 3
            + [pltpu.VMEM((CH + 16,), jnp.float32)] * 3
            + [pltpu.VMEM((CH, 128), jnp.float32)] * 3
            + [pltpu.VMEM((CH, DC), jnp.float32)]
            + [pltpu.SemaphoreType.DMA] * 4
        ),
    )(_interp_sc_body)
    return fn(i0, i1, i2, w0, w1, w2, feature2)


def _ln_relu(x, g, b):
    mu = jnp.mean(x, axis=1, keepdims=True)
    var = jnp.mean((x - mu) ** 2, axis=1, keepdims=True)
    return jnp.maximum((x - mu) / jnp.sqrt(var + 1e-5) * g + b, 0.0)


def _mlp_body(f1_ref, it_ref, w1_ref, b1_ref, g1_ref, be1_ref,
              w2_ref, b2_ref, g2_ref, be2_ref, o_ref):
    f1 = f1_ref[...]
    it = it_ref[...]
    x = (jnp.dot(f1, w1_ref[0:DF, :], preferred_element_type=jnp.float32)
         + jnp.dot(it, w1_ref[DF:DF + DC, :], preferred_element_type=jnp.float32)
         + b1_ref[...])
    x = _ln_relu(x, g1_ref[...], be1_ref[...])
    x = jnp.dot(x, w2_ref[...], preferred_element_type=jnp.float32) + b2_ref[...]
    o_ref[...] = _ln_relu(x, g2_ref[...], be2_ref[...])


def _phase3(feature1, interp, W1, b1, g1, be1, W2, b2, g2, be2,
            interpret=False):
    full = lambda a, b: pl.BlockSpec((a, b), lambda i: (0, 0))
    return pl.pallas_call(
        _mlp_body,
        grid=(N1 // R,),
        in_specs=[
            pl.BlockSpec((R, DF), lambda i: (i, 0)),
            pl.BlockSpec((R, DC), lambda i: (i, 0)),
            full(DF + DC, 64), full(1, 64), full(1, 64), full(1, 64),
            full(64, 64), full(1, 64), full(1, 64), full(1, 64),
        ],
        out_specs=pl.BlockSpec((R, 64), lambda i: (i, 0)),
        out_shape=jax.ShapeDtypeStruct((N1, 64), jnp.float32),
        interpret=interpret,
    )(feature1, interp, W1, b1.reshape(1, -1), g1.reshape(1, -1),
      be1.reshape(1, -1), W2, b2.reshape(1, -1), g2.reshape(1, -1),
      be2.reshape(1, -1))


def kernel(xyz1, xyz2, feature1, feature2, offset1, offset2,
           W1, b1, g1, be1, W2, b2, g2, be2):
    i0, i1, i2, w0, w1, w2 = _phase1(xyz1.T, xyz2)
    feature2p = jnp.pad(feature2, ((0, 0), (0, 128 - DC)))
    interp = _phase2_sc(i0, i1, i2, w0, w1, w2, feature2p)
    return _phase3(feature1, interp, W1, b1, g1, be1, W2, b2, g2, be2)
